# baseline (speedup 1.0000x reference)
import jax
from jax import lax
import jax.numpy as jnp
from jax.experimental import pallas as pl
from jax.experimental.pallas import tpu as pltpu
from jax.experimental.pallas import tpu_sc as plsc

LANES = 16
CH = 8
NBUF = 4


def _sc_gather_add(xs, times_flat, pe):
    L, B, dim = xs.shape
    n_workers = 32
    lw = (L * B) // n_workers
    nc = lw // CH
    wpb = n_workers // B

    mesh = plsc.VectorSubcoreMesh(core_axis_name="core", subcore_axis_name="subcore")

    scratch = (
        [pltpu.VMEM((lw,), jnp.int32)]
        + [pltpu.VMEM((CH, dim), jnp.float32) for _ in range(2 * NBUF)]
        + [pltpu.SemaphoreType.DMA for _ in range(3 * NBUF)]
    )

    @pl.kernel(
        out_type=jax.ShapeDtypeStruct((L, B, dim), jnp.float32),
        mesh=mesh,
        scratch_types=scratch,
    )
    def k(xs_hbm, t_hbm, pe_hbm, o_hbm, idx_v, *bufs):
        xb = bufs[0:NBUF]
        ab = bufs[NBUF:2 * NBUF]
        sx = bufs[2 * NBUF:3 * NBUF]
        sp = bufs[3 * NBUF:4 * NBUF]
        so = bufs[4 * NBUF:5 * NBUF]

        wid = lax.axis_index("core") * 16 + lax.axis_index("subcore")
        b = wid // wpb
        l_base = (wid % wpb) * lw

        pltpu.sync_copy(t_hbm.at[pl.ds(b * L + l_base, lw)], idx_v)

        def issue_loads(c, j):
            l0 = l_base + c * CH
            pltpu.async_copy(xs_hbm.at[pl.ds(l0, CH), b, :], xb[j], sx[j])
            pltpu.async_copy(pe_hbm.at[idx_v.at[pl.ds(c * CH, CH)]], ab[j], sp[j])

        def wait_loads(c, j):
            l0 = l_base + c * CH
            pltpu.make_async_copy(xs_hbm.at[pl.ds(l0, CH), b, :], xb[j], sx[j]).wait()
            pltpu.make_async_copy(
                pe_hbm.at[idx_v.at[pl.ds(c * CH, CH)]], ab[j], sp[j]).wait()

        def wait_store(c, j):
            l0 = l_base + c * CH
            pltpu.make_async_copy(ab[j], o_hbm.at[pl.ds(l0, CH), b, :], so[j]).wait()

        for j in range(2):
            issue_loads(j, j)

        @pl.loop(0, nc, step=NBUF)
        def _(cbase):
            for j in range(NBUF):
                c = cbase + j
                wait_loads(c, j)

                @pl.loop(0, CH)
                def _(r):
                    for cc in range(0, dim, LANES):
                        plsc.addupdate(
                            ab[j].at[r, pl.ds(cc, LANES)],
                            xb[j][r, pl.ds(cc, LANES)],
                        )

                l0 = l_base + c * CH
                pltpu.async_copy(ab[j], o_hbm.at[pl.ds(l0, CH), b, :], so[j])

                @pl.when(c + 2 < nc)
                def _():
                    @pl.when(c >= 2)
                    def _():
                        wait_store(c - 2, (j + 2) % NBUF)
                    issue_loads(c + 2, (j + 2) % NBUF)

        for j in range(NBUF):
            wait_store(nc - NBUF + j, j)

    return k(xs, times_flat, pe)


def kernel(xs, times, pe):
    L, B, dim = xs.shape
    times_flat = times.astype(jnp.int32).reshape(B * L)
    return _sc_gather_add(xs, times_flat, pe)
